# initial kernel scaffold (unmeasured)
import jax
import jax.numpy as jnp
from jax import lax
from jax.experimental import pallas as pl
from jax.experimental.pallas import tpu as pltpu


def kernel(
    x,
):
    def body(*refs):
        pass

    out_shape = jax.ShapeDtypeStruct(..., jnp.float32)
    return pl.pallas_call(body, out_shape=out_shape)(...)


# baseline (device time: 16135 ns/iter reference)
import jax
import jax.numpy as jnp
from jax import lax
from jax.experimental import pallas as pl
from jax.experimental.pallas import tpu as pltpu

Z = 4


def kernel(x):
    m, n = x.shape
    blk = n // Z

    def body(x_ref, out_ref, send_sems, recv_sems):
        my_x = lax.axis_index("x")
        my_y = lax.axis_index("y")
        my_z = lax.axis_index("z")

        barrier_sem = pltpu.get_barrier_semaphore()
        for r in range(1, Z):
            peer = (my_z + r) % Z
            pl.semaphore_signal(
                barrier_sem, inc=1,
                device_id=(my_x, my_y, peer),
                device_id_type=pl.DeviceIdType.MESH,
            )
        pl.semaphore_wait(barrier_sem, Z - 1)

        out_ref[pl.ds(my_z * m, m), :] = x_ref[:, pl.ds(my_z * blk, blk)]

        rdmas = []
        for r in range(1, Z):
            tgt = (my_z + r) % Z
            rdma = pltpu.make_async_remote_copy(
                src_ref=x_ref.at[:, pl.ds(tgt * blk, blk)],
                dst_ref=out_ref.at[pl.ds(my_z * m, m), :],
                send_sem=send_sems.at[r - 1],
                recv_sem=recv_sems.at[r - 1],
                device_id=(my_x, my_y, tgt),
                device_id_type=pl.DeviceIdType.MESH,
            )
            rdma.start()
            rdmas.append(rdma)
        for rdma in rdmas:
            rdma.wait()

    out_shape = jax.ShapeDtypeStruct((Z * m, blk), x.dtype)
    return pl.pallas_call(
        body,
        out_shape=out_shape,
        in_specs=[pl.BlockSpec(memory_space=pltpu.VMEM)],
        out_specs=pl.BlockSpec(memory_space=pltpu.VMEM),
        scratch_shapes=[
            pltpu.SemaphoreType.DMA((Z - 1,)),
            pltpu.SemaphoreType.DMA((Z - 1,)),
        ],
        compiler_params=pltpu.CompilerParams(collective_id=0),
    )(x)
